# baseline (speedup 1.0000x reference)
import functools

import jax
import jax.numpy as jnp
from jax import lax
from jax.experimental import pallas as pl
from jax.experimental.pallas import tpu as pltpu
from jax.experimental.pallas import tpu_sc as plsc

VOCAB = 1000000
EMB = 64
BATCH = 16384
FIELDS = 5
LANES = 16

NC = 2
NS = 16
NW = NC * NS
BPW = BATCH // NW
G = 128
NCHUNK = BPW // G
ROWS = G * FIELDS
NGRP = G // LANES
IDXR = BPW * FIELDS // G


_GDN = lax.GatherDimensionNumbers(
    offset_dims=(), collapsed_slice_dims=(0,), start_index_map=(0,))


def _bcast(vec, lane):
    idx = jnp.full((LANES, 1), lane, jnp.int32)
    return lax.gather(vec, idx, _GDN, (1,),
                      mode=lax.GatherScatterMode.PROMISE_IN_BOUNDS)


def _body(idx_hbm, table_hbm, aux_hbm, out_hbm, idx_v, rows_v, out_v, aux_v,
          gsem):
    wid = lax.axis_index("s") * NC + lax.axis_index("c")
    pltpu.sync_copy(aux_hbm, aux_v)
    pltpu.sync_copy(idx_hbm.at[wid], idx_v)
    wpat = [aux_v[k, :] for k in range(FIELDS)]
    bias = aux_v[FIELDS, :]
    gate = aux_v[FIELDS + 1, :]

    for i in range(NCHUNK):
        cps = [pltpu.async_copy(table_hbm.at[idx_v.at[i * FIELDS + k]],
                                rows_v.at[k], gsem)
               for k in range(FIELDS)]
        for cp in cps:
            cp.wait()

        def grp(g, carry):
            base = LANES * FIELDS * g
            weff = []
            for k in range(FIELDS):
                p = i * ROWS + base + LANES * k
                ik = idx_v[p >> 7, pl.ds(p & 127, LANES)]
                weff.append(jnp.where(ik == 0, 0.0, wpat[k]))
            for b in range(LANES):
                acc = [None] * (EMB // LANES)
                for c in range(FIELDS):
                    p = FIELDS * b + c
                    wv = _bcast(weff[p >> 4], p & 15)
                    q = base + p
                    for m in range(EMB // LANES):
                        r = rows_v[q >> 7, q & 127, pl.ds(LANES * m, LANES)]
                        t = wv * r
                        acc[m] = t if acc[m] is None else acc[m] + t
                gb = LANES * g + b
                for m in range(EMB // LANES):
                    o = jnp.maximum(acc[m] + bias, 0.0) * gate
                    out_v[gb, pl.ds(LANES * m, LANES)] = o
            return carry

        lax.fori_loop(0, NGRP, grp, 0)
        pltpu.sync_copy(out_v, out_hbm.at[pl.ds(wid * BPW + i * G, G)])


_mesh = plsc.VectorSubcoreMesh(core_axis_name="c", subcore_axis_name="s")

_encoder = functools.partial(
    pl.kernel,
    out_type=jax.ShapeDtypeStruct((BATCH, EMB), jnp.float32),
    mesh=_mesh,
    scratch_types=[
        pltpu.VMEM((IDXR, G), jnp.int32),
        pltpu.VMEM((FIELDS, G, EMB), jnp.float32),
        pltpu.VMEM((G, EMB), jnp.float32),
        pltpu.VMEM((FIELDS + 2, LANES), jnp.float32),
        pltpu.SemaphoreType.DMA,
    ],
    compiler_params=pltpu.CompilerParams(use_tc_tiling_on_sc=False),
)(_body)


def kernel(input, table, conv_w, conv_b, isBatched):
    idx = input.astype(jnp.int32).reshape(NW, IDXR, G)
    w = conv_w.reshape(FIELDS)
    pos = jnp.arange(FIELDS * LANES, dtype=jnp.int32) % FIELDS
    wpat = jnp.take(w, pos).reshape(FIELDS, LANES)
    bias = jnp.full((1, LANES), conv_b[0], jnp.float32)
    gate = jnp.full((1, LANES), jnp.asarray(isBatched, jnp.float32))
    aux = jnp.concatenate([wpat, bias, gate], axis=0)
    return _encoder(idx, table, aux)


# speedup vs baseline: 1.2070x; 1.2070x over previous
import functools

import jax
import jax.numpy as jnp
from jax import lax
from jax.experimental import pallas as pl
from jax.experimental.pallas import tpu as pltpu
from jax.experimental.pallas import tpu_sc as plsc

VOCAB = 1000000
EMB = 64
PITCH = 128
BATCH = 16384
FIELDS = 5
LANES = 16

NC = 2
NS = 16
NW = NC * NS
BPW = BATCH // NW
G = 128
NCHUNK = BPW // G
ROWS = G * FIELDS
NGRP = G // LANES
IDXR = BPW * FIELDS // G

_GDN = lax.GatherDimensionNumbers(
    offset_dims=(), collapsed_slice_dims=(0,), start_index_map=(0,))


def _bcast(vec, lane):
    idx = jnp.full((LANES, 1), lane, jnp.int32)
    return lax.gather(vec, idx, _GDN, (1,),
                      mode=lax.GatherScatterMode.PROMISE_IN_BOUNDS)


def _body(idx_hbm, table_hbm, aux_hbm, out_hbm, idx_v, rows_v, out_v, aux_v,
          gsem):
    wid = lax.axis_index("s") * NC + lax.axis_index("c")
    pltpu.sync_copy(aux_hbm, aux_v)
    pltpu.sync_copy(idx_hbm.at[wid], idx_v)
    wpat = [aux_v[k, pl.ds(0, LANES)] for k in range(FIELDS)]
    bias = aux_v[FIELDS, pl.ds(0, LANES)]
    gate = aux_v[FIELDS + 1, pl.ds(0, LANES)]

    for i in range(NCHUNK):
        cps = [pltpu.async_copy(table_hbm.at[idx_v.at[i * FIELDS + k]],
                                rows_v.at[pl.ds(k * G, G)], gsem)
               for k in range(FIELDS)]
        for cp in cps:
            cp.wait()

        def grp(g, carry):
            base = LANES * FIELDS * g
            weff = []
            for k in range(FIELDS):
                p = i * ROWS + base + LANES * k
                ik = idx_v[p >> 7, pl.ds(p & 127, LANES)]
                weff.append(jnp.where(ik == 0, 0.0, wpat[k]))
            for b in range(LANES):
                acc = [None] * (EMB // LANES)
                for c in range(FIELDS):
                    p = FIELDS * b + c
                    wv = _bcast(weff[p >> 4], p & 15)
                    q = base + p
                    for m in range(EMB // LANES):
                        r = rows_v[q, pl.ds(LANES * m, LANES)]
                        t = wv * r
                        acc[m] = t if acc[m] is None else acc[m] + t
                gb = LANES * g + b
                for m in range(EMB // LANES):
                    o = jnp.maximum(acc[m] + bias, 0.0) * gate
                    out_v[gb, pl.ds(LANES * m, LANES)] = o
            return carry

        lax.fori_loop(0, NGRP, grp, 0)
        pltpu.sync_copy(out_v, out_hbm.at[pl.ds(wid * BPW + i * G, G)])


_mesh = plsc.VectorSubcoreMesh(core_axis_name="c", subcore_axis_name="s")

_encoder = functools.partial(
    pl.kernel,
    out_type=jax.ShapeDtypeStruct((BATCH, PITCH), jnp.float32),
    mesh=_mesh,
    scratch_types=[
        pltpu.VMEM((IDXR, G), jnp.int32),
        pltpu.VMEM((ROWS, PITCH), jnp.float32),
        pltpu.VMEM((G, PITCH), jnp.float32),
        pltpu.VMEM((8, PITCH), jnp.float32),
        pltpu.SemaphoreType.DMA,
    ],
    compiler_params=pltpu.CompilerParams(use_tc_tiling_on_sc=True),
)(_body)


B2 = 2048


def _transpose_body(tt_ref, out_ref):
    out_ref[...] = jnp.concatenate(
        [tt_ref[...].T, jnp.zeros((B2, PITCH - EMB), jnp.float32)], axis=1)


_detile = pl.pallas_call(
    _transpose_body,
    grid=((VOCAB + B2 - 1) // B2,),
    in_specs=[pl.BlockSpec((EMB, B2), lambda i: (0, i))],
    out_specs=pl.BlockSpec((B2, PITCH), lambda i: (i, 0)),
    out_shape=jax.ShapeDtypeStruct((VOCAB, PITCH), jnp.float32),
)


def kernel(input, table, conv_w, conv_b, isBatched):
    idx = input.astype(jnp.int32).reshape(NW, IDXR, G)
    tpad = _detile(table.T)
    w = conv_w.reshape(FIELDS)
    pos = jnp.arange(FIELDS * LANES, dtype=jnp.int32) % FIELDS
    wpat = jnp.take(w, pos).reshape(FIELDS, LANES)
    bias = jnp.full((1, LANES), conv_b[0], jnp.float32)
    gate = jnp.full((1, LANES), jnp.asarray(isBatched, jnp.float32))
    aux7 = jnp.concatenate([wpat, bias, gate], axis=0)
    aux = jnp.zeros((8, PITCH), jnp.float32).at[:7, :LANES].set(aux7)
    out = _encoder(idx, tpad, aux)
    return out[:, :EMB]
